# baseline (speedup 1.0000x reference)
import functools

import jax
import jax.numpy as jnp
import numpy as np
from jax.experimental import pallas as pl
from jax.experimental.pallas import tpu as pltpu

N_ROWS = 1000000
D = 32
BLK = 32768
NSTEPS = -(-N_ROWS // BLK)
K = 16
BIG_I = np.int32(1 << 30)
INF = np.float32(np.inf)


def _body(data_ref, mean_ref, std_ref, w1t_ref, b_ref, exp_ref, mem_ref,
          out_ref, ds_ref, enc_ref, sm_ref):
    i = pl.program_id(0)

    @pl.when(i == 0)
    def _init():
        sm_ref[...] = jnp.full((1, 32), INF, jnp.float32)
        dn = (data_ref[...] - mean_ref[...]) / std_ref[...]
        dn = jnp.where(std_ref[...] == 0.0, 0.0, dn)
        p = w1t_ref[...] * dn
        enc_ref[...] = jnp.tanh(jnp.sum(p, axis=1, keepdims=True)
                                + b_ref[...])

    t = jnp.abs(mem_ref[...] - enc_ref[...])
    dist = jnp.sum(t, axis=0, keepdims=True)
    l_io = jax.lax.broadcasted_iota(jnp.int32, (1, BLK), 1)
    dist = jnp.where(i * BLK + l_io < N_ROWS, dist, INF)
    ds_ref[pl.ds(i, 1), :] = dist
    l32 = jax.lax.broadcasted_iota(jnp.int32, (1, 32), 1)
    sm_ref[...] = jnp.where(l32 == i, jnp.min(dist), sm_ref[...])

    @pl.when(i == NSTEPS - 1)
    def _extract():
        sm = sm_ref[...]
        num = jnp.float32(0.0)
        for k in range(K):
            gv = jnp.min(sm)
            srow = jnp.min(jnp.where(sm == gv, l32, BIG_I))
            band = ds_ref[pl.ds(srow, 1), :]
            fm = jnp.min(jnp.where(band == gv, l_io, BIG_I))
            band = jnp.where(l_io == fm, INF, band)
            ds_ref[pl.ds(srow, 1), :] = band
            sm = jnp.where(l32 == srow, jnp.min(band), sm)
            num = num + gv * exp_ref[0, k]
        den = jnp.float32(0.0)
        for k in range(K):
            den = den + exp_ref[0, k]
        out_ref[0, 0] = num / den


@functools.partial(jax.jit, static_argnums=())
def kernel(data, mean, std, memory, W1, b1, exp):
    mean_row = mean.reshape(1, 64)
    std_row = std.reshape(1, 64)
    w1t = W1.T
    b_col = b1.reshape(D, 1)
    exp_row = exp.reshape(1, K)
    mem_t = memory.T

    const = lambda i: (0, 0)
    out = pl.pallas_call(
        _body,
        grid=(NSTEPS,),
        in_specs=[
            pl.BlockSpec((1, 64), const),
            pl.BlockSpec((1, 64), const),
            pl.BlockSpec((1, 64), const),
            pl.BlockSpec((D, 64), const),
            pl.BlockSpec((D, 1), const),
            pl.BlockSpec((1, K), const, memory_space=pltpu.SMEM),
            pl.BlockSpec((D, BLK), lambda i: (0, i)),
        ],
        out_specs=pl.BlockSpec((1, 1), const, memory_space=pltpu.SMEM),
        out_shape=jax.ShapeDtypeStruct((1, 1), jnp.float32),
        scratch_shapes=[
            pltpu.VMEM((32, BLK), jnp.float32),
            pltpu.VMEM((D, 1), jnp.float32),
            pltpu.VMEM((1, 32), jnp.float32),
        ],
        compiler_params=pltpu.CompilerParams(
            dimension_semantics=("arbitrary",)),
    )(data, mean_row, std_row, w1t, b_col, exp_row, mem_t)
    return out.reshape(())


# speedup vs baseline: 1.0884x; 1.0884x over previous
import functools

import jax
import jax.numpy as jnp
import numpy as np
from jax.experimental import pallas as pl
from jax.experimental.pallas import tpu as pltpu

N_ROWS = 1000000
D = 32
BLK = 32768
SUB = BLK // 8
NSTEPS = -(-N_ROWS // BLK)
K = 16
BIG_I = np.int32(1 << 30)
INF = np.float32(np.inf)


def _body(data_ref, mean_ref, std_ref, w1t_ref, b_ref, exp_ref, mem_ref,
          out_ref, ds_ref, enc_ref, sm_ref):
    i = pl.program_id(0)

    @pl.when(i == 0)
    def _init():
        sm_ref[...] = jnp.full((1, 32), INF, jnp.float32)
        dn = (data_ref[...] - mean_ref[...]) / std_ref[...]
        dn = jnp.where(std_ref[...] == 0.0, 0.0, dn)
        p = w1t_ref[...] * dn
        enc_ref[...] = jnp.tanh(jnp.sum(p, axis=1, keepdims=True)
                                + b_ref[...])

    t = jnp.abs(mem_ref[...] - enc_ref[...])
    dist = jnp.sum(t, axis=0, keepdims=True)
    l_io = jax.lax.broadcasted_iota(jnp.int32, (1, BLK), 1)
    dist = jnp.where(i * BLK + l_io < N_ROWS, dist, INF)
    for s in range(8):
        ds_ref[pl.ds(i * 8 + s, 1), :] = dist[:, s * SUB:(s + 1) * SUB]
    l32 = jax.lax.broadcasted_iota(jnp.int32, (1, 32), 1)
    sm_ref[...] = jnp.where(l32 == i, jnp.min(dist), sm_ref[...])

    @pl.when(i == NSTEPS - 1)
    def _extract():
        sm = sm_ref[...]
        b_io = (jax.lax.broadcasted_iota(jnp.int32, (8, SUB), 0) * SUB
                + jax.lax.broadcasted_iota(jnp.int32, (8, SUB), 1))
        num = jnp.float32(0.0)
        for k in range(K):
            gv = jnp.min(sm)
            srow = jnp.min(jnp.where(sm == gv, l32, BIG_I))
            band = ds_ref[pl.ds(srow * 8, 8), :]
            fm = jnp.min(jnp.where(band == gv, b_io, BIG_I))
            band = jnp.where(b_io == fm, INF, band)
            ds_ref[pl.ds(srow * 8, 8), :] = band
            sm = jnp.where(l32 == srow, jnp.min(band), sm)
            num = num + gv * exp_ref[0, k]
        den = jnp.float32(0.0)
        for k in range(K):
            den = den + exp_ref[0, k]
        out_ref[0, 0] = num / den


@functools.partial(jax.jit, static_argnums=())
def kernel(data, mean, std, memory, W1, b1, exp):
    mean_row = mean.reshape(1, 64)
    std_row = std.reshape(1, 64)
    w1t = W1.T
    b_col = b1.reshape(D, 1)
    exp_row = exp.reshape(1, K)
    mem_t = memory.T

    const = lambda i: (0, 0)
    out = pl.pallas_call(
        _body,
        grid=(NSTEPS,),
        in_specs=[
            pl.BlockSpec((1, 64), const),
            pl.BlockSpec((1, 64), const),
            pl.BlockSpec((1, 64), const),
            pl.BlockSpec((D, 64), const),
            pl.BlockSpec((D, 1), const),
            pl.BlockSpec((1, K), const, memory_space=pltpu.SMEM),
            pl.BlockSpec((D, BLK), lambda i: (0, i)),
        ],
        out_specs=pl.BlockSpec((1, 1), const, memory_space=pltpu.SMEM),
        out_shape=jax.ShapeDtypeStruct((1, 1), jnp.float32),
        scratch_shapes=[
            pltpu.VMEM((NSTEPS * 8, SUB), jnp.float32),
            pltpu.VMEM((D, 1), jnp.float32),
            pltpu.VMEM((1, 32), jnp.float32),
        ],
        compiler_params=pltpu.CompilerParams(
            dimension_semantics=("arbitrary",)),
    )(data, mean_row, std_row, w1t, b_col, exp_row, mem_t)
    return out.reshape(())


# speedup vs baseline: 1.2235x; 1.1241x over previous
import functools

import jax
import jax.numpy as jnp
import numpy as np
from jax.experimental import pallas as pl
from jax.experimental.pallas import tpu as pltpu

N_ROWS = 1000000
D = 32
BLK = 32768
SUB = BLK // 8
NBLK = 31
NSTEPS = 16
K = 16
BIG_I = np.int32(1 << 30)
INF = np.float32(np.inf)


def _body(data_ref, mean_ref, std_ref, w1t_ref, b_ref, exp_ref,
          mem_a_ref, mem_b_ref, out_ref, ds_ref, enc_ref, sm_ref):
    i = pl.program_id(0)

    @pl.when(i == 0)
    def _init():
        sm_ref[...] = jnp.full((1, 32), INF, jnp.float32)
        dn = (data_ref[...] - mean_ref[...]) / std_ref[...]
        dn = jnp.where(std_ref[...] == 0.0, 0.0, dn)
        p = w1t_ref[...] * dn
        enc_ref[...] = jnp.tanh(jnp.sum(p, axis=1, keepdims=True)
                                + b_ref[...])

    l_io = jax.lax.broadcasted_iota(jnp.int32, (1, BLK), 1)
    l32 = jax.lax.broadcasted_iota(jnp.int32, (1, 32), 1)

    def _block(mem_ref, b):
        t = jnp.abs(mem_ref[...] - enc_ref[...])
        dist = jnp.sum(t, axis=0, keepdims=True)
        dist = jnp.where(b * BLK + l_io < N_ROWS, dist, INF)
        for s in range(8):
            ds_ref[pl.ds(b * 8 + s, 1), :] = dist[:, s * SUB:(s + 1) * SUB]
        sm_ref[...] = jnp.where(l32 == b, jnp.min(dist), sm_ref[...])

    _block(mem_a_ref, i)
    _block(mem_b_ref, 16 + jnp.minimum(i, 14))

    @pl.when(i == NSTEPS - 1)
    def _extract():
        sm = sm_ref[...]
        b_io = (jax.lax.broadcasted_iota(jnp.int32, (8, SUB), 0) * SUB
                + jax.lax.broadcasted_iota(jnp.int32, (8, SUB), 1))
        num = jnp.float32(0.0)
        for k in range(K):
            gv = jnp.min(sm)
            srow = jnp.min(jnp.where(sm == gv, l32, BIG_I))
            band = ds_ref[pl.ds(srow * 8, 8), :]
            fm = jnp.min(jnp.where(band == gv, b_io, BIG_I))
            band = jnp.where(b_io == fm, INF, band)
            ds_ref[pl.ds(srow * 8, 8), :] = band
            sm = jnp.where(l32 == srow, jnp.min(band), sm)
            num = num + gv * exp_ref[0, k]
        den = jnp.float32(0.0)
        for k in range(K):
            den = den + exp_ref[0, k]
        out_ref[0, 0] = num / den


@functools.partial(jax.jit, static_argnums=())
def kernel(data, mean, std, memory, W1, b1, exp):
    mean_row = mean.reshape(1, 64)
    std_row = std.reshape(1, 64)
    w1t = W1.T
    b_col = b1.reshape(D, 1)
    exp_row = exp.reshape(1, K)
    mem_t = memory.T

    const = lambda i: (0, 0)
    out = pl.pallas_call(
        _body,
        grid=(NSTEPS,),
        in_specs=[
            pl.BlockSpec((1, 64), const),
            pl.BlockSpec((1, 64), const),
            pl.BlockSpec((1, 64), const),
            pl.BlockSpec((D, 64), const),
            pl.BlockSpec((D, 1), const),
            pl.BlockSpec((1, K), const, memory_space=pltpu.SMEM),
            pl.BlockSpec((D, BLK), lambda i: (0, i)),
            pl.BlockSpec((D, BLK), lambda i: (0, 16 + jnp.minimum(i, 14))),
        ],
        out_specs=pl.BlockSpec((1, 1), const, memory_space=pltpu.SMEM),
        out_shape=jax.ShapeDtypeStruct((1, 1), jnp.float32),
        scratch_shapes=[
            pltpu.VMEM((NBLK * 8 + 8, SUB), jnp.float32),
            pltpu.VMEM((D, 1), jnp.float32),
            pltpu.VMEM((1, 32), jnp.float32),
        ],
        compiler_params=pltpu.CompilerParams(
            dimension_semantics=("arbitrary",)),
    )(data, mean_row, std_row, w1t, b_col, exp_row, mem_t, mem_t)
    return out.reshape(())


# speedup vs baseline: 1.3193x; 1.0784x over previous
import functools

import jax
import jax.numpy as jnp
import numpy as np
from jax.experimental import pallas as pl
from jax.experimental.pallas import tpu as pltpu

N_ROWS = 1000000
D = 32
BLK = 32768
SUB = BLK // 8
NBLK = 31
NSTEPS = 8
K = 16
BIG_I = np.int32(1 << 30)
INF = np.float32(np.inf)


def _body(data_ref, mean_ref, std_ref, w1t_ref, b_ref, exp_ref,
          mem_a_ref, mem_b_ref, mem_c_ref, mem_d_ref,
          out_ref, ds_ref, enc_ref, sm_ref):
    i = pl.program_id(0)

    @pl.when(i == 0)
    def _init():
        sm_ref[...] = jnp.full((1, 32), INF, jnp.float32)
        dn = (data_ref[...] - mean_ref[...]) / std_ref[...]
        dn = jnp.where(std_ref[...] == 0.0, 0.0, dn)
        p = w1t_ref[...] * dn
        enc_ref[...] = jnp.tanh(jnp.sum(p, axis=1, keepdims=True)
                                + b_ref[...])

    l_io = jax.lax.broadcasted_iota(jnp.int32, (1, BLK), 1)
    l32 = jax.lax.broadcasted_iota(jnp.int32, (1, 32), 1)

    def _block(mem_ref, b):
        t = jnp.abs(mem_ref[...] - enc_ref[...])
        dist = jnp.sum(t, axis=0, keepdims=True)
        dist = jnp.where(b * BLK + l_io < N_ROWS, dist, INF)
        for s in range(8):
            ds_ref[pl.ds(b * 8 + s, 1), :] = dist[:, s * SUB:(s + 1) * SUB]
        sm_ref[...] = jnp.where(l32 == b, jnp.min(dist), sm_ref[...])

    _block(mem_a_ref, i)
    _block(mem_b_ref, 8 + i)
    _block(mem_c_ref, 16 + i)
    _block(mem_d_ref, 24 + jnp.minimum(i, 6))

    @pl.when(i == NSTEPS - 1)
    def _extract():
        sm = sm_ref[...]
        b_io = (jax.lax.broadcasted_iota(jnp.int32, (8, SUB), 0) * SUB
                + jax.lax.broadcasted_iota(jnp.int32, (8, SUB), 1))
        num = jnp.float32(0.0)
        for k in range(K):
            gv = jnp.min(sm)
            srow = jnp.min(jnp.where(sm == gv, l32, BIG_I))
            band = ds_ref[pl.ds(srow * 8, 8), :]
            fm = jnp.min(jnp.where(band == gv, b_io, BIG_I))
            band = jnp.where(b_io == fm, INF, band)
            ds_ref[pl.ds(srow * 8, 8), :] = band
            sm = jnp.where(l32 == srow, jnp.min(band), sm)
            num = num + gv * exp_ref[0, k]
        den = jnp.float32(0.0)
        for k in range(K):
            den = den + exp_ref[0, k]
        out_ref[0, 0] = num / den


@functools.partial(jax.jit, static_argnums=())
def kernel(data, mean, std, memory, W1, b1, exp):
    mean_row = mean.reshape(1, 64)
    std_row = std.reshape(1, 64)
    w1t = W1.T
    b_col = b1.reshape(D, 1)
    exp_row = exp.reshape(1, K)
    mem_t = memory.T

    const = lambda i: (0, 0)
    out = pl.pallas_call(
        _body,
        grid=(NSTEPS,),
        in_specs=[
            pl.BlockSpec((1, 64), const),
            pl.BlockSpec((1, 64), const),
            pl.BlockSpec((1, 64), const),
            pl.BlockSpec((D, 64), const),
            pl.BlockSpec((D, 1), const),
            pl.BlockSpec((1, K), const, memory_space=pltpu.SMEM),
            pl.BlockSpec((D, BLK), lambda i: (0, i)),
            pl.BlockSpec((D, BLK), lambda i: (0, 8 + i)),
            pl.BlockSpec((D, BLK), lambda i: (0, 16 + i)),
            pl.BlockSpec((D, BLK), lambda i: (0, 24 + jnp.minimum(i, 6))),
        ],
        out_specs=pl.BlockSpec((1, 1), const, memory_space=pltpu.SMEM),
        out_shape=jax.ShapeDtypeStruct((1, 1), jnp.float32),
        scratch_shapes=[
            pltpu.VMEM((NBLK * 8 + 8, SUB), jnp.float32),
            pltpu.VMEM((D, 1), jnp.float32),
            pltpu.VMEM((1, 32), jnp.float32),
        ],
        compiler_params=pltpu.CompilerParams(
            dimension_semantics=("arbitrary",)),
    )(data, mean_row, std_row, w1t, b_col, exp_row, mem_t, mem_t, mem_t, mem_t)
    return out.reshape(())
